# Initial kernel scaffold; baseline (speedup 1.0000x reference)
import jax
import jax.numpy as jnp
from jax.experimental import pallas as pl


def kernel(x_raingauge, x_radar, edge_index_rr, edge_index_radar_rg, edge_index_rg_radar, edge_weight_rr, edge_weight_radar_rg, edge_weight_rg_radar, Wrel0_rr, brel0_rr, Wroot0_rr, Wrel0_mrg, brel0_mrg, Wroot0_mrg, Wrel0_rgm, brel0_rgm, Wroot0_rgm, Wrel1_rr, brel1_rr, Wroot1_rr, Wrel1_mrg, brel1_mrg, Wroot1_mrg, Wrel1_rgm, brel1_rgm, Wroot1_rgm, Wlin, blin):
    raise NotImplementedError("write your pallas kernel here")


# speedup vs baseline: 3.5922x; 3.5922x over previous
import functools

import jax
import jax.numpy as jnp
from jax import lax
from jax.experimental import pallas as pl
from jax.experimental.pallas import tpu as pltpu
from jax.experimental.pallas import tpu_sc as plsc

NC = 2
NS = 16
WIN = 128


@functools.lru_cache(maxsize=None)
def _make_segsum(n_src, n_dst, n_win, esplit):
    chunk = (-(-n_dst // NS) + 7) // 8 * 8
    last = n_dst - (NS - 1) * chunk
    mesh = plsc.VectorSubcoreMesh(
        core_axis_name="c", subcore_axis_name="s", num_cores=NC,
        num_subcores=NS)

    @functools.partial(
        pl.kernel,
        out_type=(jax.ShapeDtypeStruct((n_dst, 128), jnp.float32),
                  jax.ShapeDtypeStruct((n_dst, 128), jnp.float32)),
        mesh=mesh,
        scratch_types=[
            pltpu.VMEM((n_win, WIN), jnp.int32),
            pltpu.VMEM((n_win, WIN), jnp.int32),
            pltpu.VMEM((n_win, WIN), jnp.float32),
            pltpu.VMEM((WIN, 128), jnp.float32),
            pltpu.VMEM_SHARED((n_dst, 128), jnp.float32),
            pltpu.SemaphoreType.DMA,
        ],
    )
    def seg(x0_hbm, x1_hbm, src_hbm, dst_hbm, w_hbm, zeros_hbm, out0, out1,
            src_v, dst_v, w_v, rows_v, acc, sem):
        c = lax.axis_index("c")
        s = lax.axis_index("s")
        base = (c * NS + s) * n_win if esplit else s * n_win

        @pl.when(s < NS - 1)
        def _():
            pltpu.sync_copy(zeros_hbm.at[pl.ds(s * chunk, chunk)],
                            acc.at[pl.ds(s * chunk, chunk)])

        @pl.when(s == NS - 1)
        def _():
            pltpu.sync_copy(zeros_hbm.at[pl.ds((NS - 1) * chunk, last)],
                            acc.at[pl.ds((NS - 1) * chunk, last)])
        pltpu.sync_copy(src_hbm.at[pl.ds(base, n_win)], src_v)
        pltpu.sync_copy(dst_hbm.at[pl.ds(base, n_win)], dst_v)
        pltpu.sync_copy(w_hbm.at[pl.ds(base, n_win)], w_v)
        plsc.subcore_barrier()

        def window(w, carry):
            @pl.when(c == 0)
            def _():
                pltpu.async_copy(x0_hbm.at[src_v.at[w]], rows_v, sem).wait()

            @pl.when(c == 1)
            def _():
                pltpu.async_copy(x1_hbm.at[src_v.at[w]], rows_v, sem).wait()

            def group(g, gc):
                w16 = w_v[w, pl.ds(g * 16, 16)]
                for j in range(16):
                    wv = w16[jnp.full((16,), j, jnp.int32)]
                    r = g * 16 + j
                    for k in range(8):
                        rows_v[r, pl.ds(k * 16, 16)] = (
                            rows_v[r, pl.ds(k * 16, 16)] * wv)
                return gc

            lax.fori_loop(0, WIN // 16, group, 0)
            pltpu.sync_copy(rows_v, acc.at[dst_v.at[w]], add=True)
            return carry

        lax.fori_loop(0, n_win, window, 0)
        plsc.subcore_barrier()

        @pl.when((c == 0) & (s < NS - 1))
        def _():
            pltpu.sync_copy(acc.at[pl.ds(s * chunk, chunk)],
                            out0.at[pl.ds(s * chunk, chunk)])

        @pl.when((c == 0) & (s == NS - 1))
        def _():
            pltpu.sync_copy(acc.at[pl.ds((NS - 1) * chunk, last)],
                            out0.at[pl.ds((NS - 1) * chunk, last)])

        @pl.when((c == 1) & (s < NS - 1))
        def _():
            pltpu.sync_copy(acc.at[pl.ds(s * chunk, chunk)],
                            out1.at[pl.ds(s * chunk, chunk)])

        @pl.when((c == 1) & (s == NS - 1))
        def _():
            pltpu.sync_copy(acc.at[pl.ds((NS - 1) * chunk, last)],
                            out1.at[pl.ds((NS - 1) * chunk, last)])

    return seg


def _prep_edges(edge_index, weight, n_src, n_dst, esplit):
    e = weight.shape[0]
    workers = NC * NS if esplit else NS
    n_win = -(-(-(-e // (workers * WIN))) // 8) * 8
    per_worker = n_win * WIN
    tot = workers * per_worker
    src = edge_index[0].astype(jnp.int32)
    dst = edge_index[1].astype(jnp.int32)
    w = weight.astype(jnp.float32)
    pad = tot - e
    if pad:
        r = jnp.arange(pad, dtype=jnp.int32)
        src = jnp.concatenate([src, r % n_src])
        dst = jnp.concatenate([dst, r % n_dst])
        w = jnp.concatenate([w, jnp.zeros((pad,), jnp.float32)])
    shape = (workers * n_win, WIN)
    return src.reshape(shape), dst.reshape(shape), w.reshape(shape), n_win


_BLK = 1000


def _tc_fused(inps, ws, bias, wlin=None, blin=None, split=False):
    n = inps[0].shape[0]
    grid = (n // _BLK,)
    nin = len(inps)
    hh = ws[0].shape[1]
    proj = wlin is not None

    def body(*refs):
        arefs = refs[:nin]
        wrefs = refs[nin:2 * nin]
        brf = refs[2 * nin]
        rest = refs[2 * nin + 1:]
        acc = jnp.dot(arefs[0][...], wrefs[0][...],
                      preferred_element_type=jnp.float32)
        for a, wt in zip(arefs[1:], wrefs[1:]):
            acc = acc + jnp.dot(a[...], wt[...],
                                preferred_element_type=jnp.float32)
        z = jnp.maximum(acc + brf[...], 0.0)
        if proj:
            wl, bl, out = rest
            out[...] = jnp.dot(z, wl[...],
                               preferred_element_type=jnp.float32) + bl[...]
        elif split:
            o0, o1 = rest
            o0[...] = z[:, :128]
            o1[...] = z[:, 128:]
        else:
            rest[0][...] = z

    in_specs = (
        [pl.BlockSpec((_BLK, a.shape[1]), lambda i: (i, 0)) for a in inps]
        + [pl.BlockSpec(wt.shape, lambda i: (0, 0)) for wt in ws]
        + [pl.BlockSpec((1, hh), lambda i: (0, 0))])
    args = list(inps) + list(ws) + [bias.reshape(1, hh)]
    if proj:
        po = wlin.shape[1]
        in_specs += [pl.BlockSpec(wlin.shape, lambda i: (0, 0)),
                     pl.BlockSpec((1, po), lambda i: (0, 0))]
        args += [wlin, blin.reshape(1, po)]
        out_shape = jax.ShapeDtypeStruct((n, po), jnp.float32)
        out_specs = pl.BlockSpec((_BLK, po), lambda i: (i, 0))
    elif split:
        out_shape = (jax.ShapeDtypeStruct((n, 128), jnp.float32),
                     jax.ShapeDtypeStruct((n, 128), jnp.float32))
        out_specs = (pl.BlockSpec((_BLK, 128), lambda i: (i, 0)),
                     pl.BlockSpec((_BLK, 128), lambda i: (i, 0)))
    else:
        out_shape = jax.ShapeDtypeStruct((n, hh), jnp.float32)
        out_specs = pl.BlockSpec((_BLK, hh), lambda i: (i, 0))

    return pl.pallas_call(
        body, grid=grid, in_specs=in_specs, out_specs=out_specs,
        out_shape=out_shape)(*args)


def kernel(x_raingauge, x_radar, edge_index_rr, edge_index_radar_rg,
           edge_index_rg_radar, edge_weight_rr, edge_weight_radar_rg,
           edge_weight_rg_radar, Wrel0_rr, brel0_rr, Wroot0_rr, Wrel0_mrg,
           brel0_mrg, Wroot0_mrg, Wrel0_rgm, brel0_rgm, Wroot0_rgm, Wrel1_rr,
           brel1_rr, Wroot1_rr, Wrel1_mrg, brel1_mrg, Wroot1_mrg, Wrel1_rgm,
           brel1_rgm, Wroot1_rgm, Wlin, blin):
    n_rg = x_raingauge.shape[0]
    n_m = x_radar.shape[0]

    zeros = jnp.zeros((n_rg, 128), jnp.float32)
    x_rg0 = x_raingauge[:, :128]
    x_rg1 = x_raingauge[:, 128:]

    src_rr, dst_rr, w_rr, nw_rr = _prep_edges(
        edge_index_rr, edge_weight_rr, n_rg, n_rg, False)
    src_mrg, dst_mrg, w_mrg, nw_mrg = _prep_edges(
        edge_index_radar_rg, edge_weight_radar_rg, n_m, n_rg, True)
    src_mrg2, dst_mrg2, w_mrg2, nw_mrg2 = _prep_edges(
        edge_index_radar_rg, edge_weight_radar_rg, n_m, n_rg, False)
    src_rgm, dst_rgm, w_rgm, nw_rgm = _prep_edges(
        edge_index_rg_radar, edge_weight_rg_radar, n_rg, n_m, False)

    seg_f_rr = _make_segsum(n_rg, n_rg, nw_rr, False)
    seg_e_mrg = _make_segsum(n_m, n_rg, nw_mrg, True)
    seg_f_mrg = _make_segsum(n_m, n_rg, nw_mrg2, False)
    seg_f_rgm = _make_segsum(n_rg, n_m, nw_rgm, False)

    a0rr0, a0rr1 = seg_f_rr(x_rg0, x_rg1, src_rr, dst_rr, w_rr, zeros)
    p0, p1 = seg_e_mrg(x_radar, x_radar, src_mrg, dst_mrg, w_mrg, zeros)
    a0gm0, a0gm1 = seg_f_rgm(x_rg0, x_rg1, src_rgm, dst_rgm, w_rgm, zeros)

    h_rg0, h_rg1 = _tc_fused(
        [a0rr0, a0rr1, x_raingauge, p0, p1],
        [Wrel0_rr[:128], Wrel0_rr[128:], Wroot0_rr + Wroot0_mrg,
         Wrel0_mrg, Wrel0_mrg],
        brel0_rr + brel0_mrg, split=True)
    h_m0, h_m1 = _tc_fused(
        [a0gm0, a0gm1, x_radar],
        [Wrel0_rgm[:128], Wrel0_rgm[128:], Wroot0_rgm],
        brel0_rgm, split=True)

    a1rr0, a1rr1 = seg_f_rr(h_rg0, h_rg1, src_rr, dst_rr, w_rr, zeros)
    a1mg0, a1mg1 = seg_f_mrg(h_m0, h_m1, src_mrg2, dst_mrg2, w_mrg2, zeros)

    wroot1 = Wroot1_rr + Wroot1_mrg
    out = _tc_fused(
        [a1rr0, a1rr1, h_rg0, h_rg1, a1mg0, a1mg1],
        [Wrel1_rr[:128], Wrel1_rr[128:], wroot1[:128], wroot1[128:],
         Wrel1_mrg[:128], Wrel1_mrg[128:]],
        brel1_rr + brel1_mrg, wlin=Wlin, blin=blin)
    return out
